# baseline (speedup 1.0000x reference)
import functools

import jax
import jax.numpy as jnp
from jax import lax
from jax.experimental import pallas as pl
from jax.experimental.pallas import tpu as pltpu
from jax.experimental.pallas import tpu_sc as plsc

B = 4096
S = 200
D = 64
H = 256
CA = 104
CB = 96

NC = 2
NS = 16
NW = NC * NS
ROWS_PER_W = B // NW
CHUNKS_PER_W = 2 * ROWS_PER_W
NBUF = 8


def _sc_pool(x, emb):
  mesh = plsc.VectorSubcoreMesh(core_axis_name="c", subcore_axis_name="s")

  @functools.partial(
      pl.kernel,
      out_type=jax.ShapeDtypeStruct((B, D), jnp.float32),
      mesh=mesh,
      compiler_params=pltpu.CompilerParams(use_tc_tiling_on_sc=False),
      scratch_types=[
          pltpu.VMEM((ROWS_PER_W, S), jnp.int32),
          pltpu.VMEM((CA, D), jnp.float32),
          pltpu.VMEM((CA, D), jnp.float32),
          pltpu.VMEM((CA, D), jnp.float32),
          pltpu.VMEM((CA, D), jnp.float32),
          pltpu.VMEM((CA, D), jnp.float32),
          pltpu.VMEM((CA, D), jnp.float32),
          pltpu.VMEM((CA, D), jnp.float32),
          pltpu.VMEM((CA, D), jnp.float32),
          pltpu.VMEM((ROWS_PER_W, D), jnp.float32),
          pltpu.SemaphoreType.DMA,
          pltpu.SemaphoreType.DMA,
          pltpu.SemaphoreType.DMA,
          pltpu.SemaphoreType.DMA,
          pltpu.SemaphoreType.DMA,
          pltpu.SemaphoreType.DMA,
          pltpu.SemaphoreType.DMA,
          pltpu.SemaphoreType.DMA,
      ],
  )
  def pool(x_hbm, emb_hbm, out_hbm, idx_v, r0, r1, r2, r3, r4, r5, r6, r7,
           out_v, s0, s1, s2, s3, s4, s5, s6, s7):
    bufs = (r0, r1, r2, r3, r4, r5, r6, r7)
    sems = (s0, s1, s2, s3, s4, s5, s6, s7)
    wid = lax.axis_index("s") * NC + lax.axis_index("c")

    pltpu.sync_copy(x_hbm.at[pl.ds(wid * ROWS_PER_W, ROWS_PER_W)], idx_v)

    def chunk_idx(j, even):
      r = j >> 1
      if even:
        return idx_v.at[r, pl.ds(0, CA)]
      return idx_v.at[r, pl.ds(CA, CB)]

    def fire(j, b):
      even = (b % 2 == 0)
      n = CA if even else CB
      pltpu.async_copy(emb_hbm.at[chunk_idx(j, even)],
                       bufs[b].at[pl.ds(0, n)], sems[b])

    def wait(j, b):
      even = (b % 2 == 0)
      n = CA if even else CB
      pltpu.make_async_copy(emb_hbm.at[chunk_idx(j, even)],
                            bufs[b].at[pl.ds(0, n)], sems[b]).wait()

    def accum(buf, n, acc):
      def body(s, acc):
        a0, a1, a2, a3 = acc
        s4 = s * 4
        for u in range(4):
          r = s4 + u
          a0 = a0 + buf[r, pl.ds(0, 16)]
          a1 = a1 + buf[r, pl.ds(16, 16)]
          a2 = a2 + buf[r, pl.ds(32, 16)]
          a3 = a3 + buf[r, pl.ds(48, 16)]
        return (a0, a1, a2, a3)
      return lax.fori_loop(0, n // 4, body, acc)

    for b in range(NBUF):
      fire(b, b)

    def process(j0, last):
      for pair in range(NBUF // 2):
        zero = jnp.zeros((16,), jnp.float32)
        acc = (zero, zero, zero, zero)
        for h in range(2):
          b = 2 * pair + h
          j = j0 + b
          wait(j, b)
          acc = accum(bufs[b], CA if h == 0 else CB, acc)
          if not last:
            fire(j + NBUF, b)
        r = (j0 >> 1) + pair
        for k in range(4):
          out_v[r, pl.ds(k * 16, 16)] = acc[k]

    @pl.loop(0, CHUNKS_PER_W - NBUF, step=NBUF)
    def _(j0):
      process(j0, last=False)

    process(CHUNKS_PER_W - NBUF, last=True)

    pltpu.sync_copy(out_v, out_hbm.at[pl.ds(wid * ROWS_PER_W, ROWS_PER_W)])

  return pool(x, emb)


TBLK = 16384
NBLK = -(-1000000 // TBLK)
VPAD = NBLK * TBLK
HSHIFT = (TBLK // 2).bit_length() - 1


def _detile(embT):
  def body(in_ref, o_ref):
    arr = in_ref[...]
    stacked = jnp.concatenate(
        [arr[:, : TBLK // 2], arr[:, TBLK // 2 :]], axis=0)
    eye = (jax.lax.broadcasted_iota(jnp.int32, (2 * D, 2 * D), 0) ==
           jax.lax.broadcasted_iota(jnp.int32, (2 * D, 2 * D), 1)
           ).astype(jnp.float32)
    o_ref[...] = lax.dot_general(
        stacked, eye, (((0,), (0,)), ((), ())),
        preferred_element_type=jnp.float32,
        precision=lax.Precision.HIGHEST)

  return pl.pallas_call(
      body,
      grid=(NBLK,),
      in_specs=[pl.BlockSpec((D, TBLK), lambda b: (0, b))],
      out_specs=pl.BlockSpec((TBLK // 2, 2 * D), lambda b: (b, 0)),
      out_shape=jax.ShapeDtypeStruct((VPAD // 2, 2 * D), jnp.float32),
  )(embT)


def _mlp(x, sums, emb0, W1, b1, W2, b2):
  BLK = 512

  def body(x_ref, s_ref, e0_ref, w1_ref, b1_ref, w2_ref, b2_ref, o_ref):
    xm = (x_ref[...] != 0).astype(jnp.float32)
    ln = jnp.sum(xm, axis=1, keepdims=True)
    avg = (s_ref[...] - (S - ln) * e0_ref[...]) / ln
    hdn = lax.dot_general(avg, w1_ref[...], (((1,), (1,)), ((), ())),
                          preferred_element_type=jnp.float32) + b1_ref[...]
    hdn = jnp.maximum(hdn, 0.0)
    logits = lax.dot_general(hdn, w2_ref[...], (((1,), (1,)), ((), ())),
                             preferred_element_type=jnp.float32) + b2_ref[...]
    m = jnp.max(logits, axis=1, keepdims=True)
    lse = m + jnp.log(jnp.sum(jnp.exp(logits - m), axis=1, keepdims=True))
    o_ref[...] = logits - lse

  return pl.pallas_call(
      body,
      grid=(B // BLK,),
      in_specs=[
          pl.BlockSpec((BLK, S), lambda i: (i, 0)),
          pl.BlockSpec((BLK, D), lambda i: (i, 0)),
          pl.BlockSpec((1, D), lambda i: (0, 0)),
          pl.BlockSpec((H, D), lambda i: (0, 0)),
          pl.BlockSpec((1, H), lambda i: (0, 0)),
          pl.BlockSpec((2, H), lambda i: (0, 0)),
          pl.BlockSpec((1, 2), lambda i: (0, 0)),
      ],
      out_specs=pl.BlockSpec((BLK, 2), lambda i: (i, 0)),
      out_shape=jax.ShapeDtypeStruct((B, 2), jnp.float32),
  )(x, sums, emb0, W1, b1, W2, b2)


def kernel(x, emb, W1, b1, W2, b2):
  x = x.astype(jnp.int32)
  emb_lin = _detile(emb.T).reshape(VPAD, D)
  e = x & (TBLK - 1)
  xr = (x - e) + ((e << 1) & (TBLK - 1)) + (e >> HSHIFT)
  sums = _sc_pool(xr, emb_lin)
  emb0 = lax.slice(emb, (0, 0), (1, D))
  return _mlp(x, sums, emb0, W1, b1.reshape(1, H), W2, b2.reshape(1, 2))


# speedup vs baseline: 1.0708x; 1.0708x over previous
import functools

import jax
import jax.numpy as jnp
from jax import lax
from jax.experimental import pallas as pl
from jax.experimental.pallas import tpu as pltpu
from jax.experimental.pallas import tpu_sc as plsc

B = 4096
S = 200
D = 64
H = 256
CA = 104
CB = 96

NC = 2
NS = 16
NW = NC * NS
ROWS_PER_W = B // NW
CHUNKS_PER_W = 2 * ROWS_PER_W
NBUF = 8


def _sc_pool(x, emb):
  mesh = plsc.VectorSubcoreMesh(core_axis_name="c", subcore_axis_name="s")

  @functools.partial(
      pl.kernel,
      out_type=jax.ShapeDtypeStruct((B, D), jnp.float32),
      mesh=mesh,
      compiler_params=pltpu.CompilerParams(use_tc_tiling_on_sc=False),
      scratch_types=[
          pltpu.VMEM((ROWS_PER_W, S), jnp.int32),
          pltpu.VMEM((CA, D), jnp.float32),
          pltpu.VMEM((CA, D), jnp.float32),
          pltpu.VMEM((CA, D), jnp.float32),
          pltpu.VMEM((CA, D), jnp.float32),
          pltpu.VMEM((CA, D), jnp.float32),
          pltpu.VMEM((CA, D), jnp.float32),
          pltpu.VMEM((CA, D), jnp.float32),
          pltpu.VMEM((CA, D), jnp.float32),
          pltpu.VMEM((ROWS_PER_W, D), jnp.float32),
          pltpu.SemaphoreType.DMA,
          pltpu.SemaphoreType.DMA,
          pltpu.SemaphoreType.DMA,
          pltpu.SemaphoreType.DMA,
          pltpu.SemaphoreType.DMA,
          pltpu.SemaphoreType.DMA,
          pltpu.SemaphoreType.DMA,
          pltpu.SemaphoreType.DMA,
      ],
  )
  def pool(x_hbm, emb_hbm, out_hbm, idx_v, r0, r1, r2, r3, r4, r5, r6, r7,
           out_v, s0, s1, s2, s3, s4, s5, s6, s7):
    bufs = (r0, r1, r2, r3, r4, r5, r6, r7)
    sems = (s0, s1, s2, s3, s4, s5, s6, s7)
    wid = lax.axis_index("s") * NC + lax.axis_index("c")

    pltpu.sync_copy(x_hbm.at[pl.ds(wid * ROWS_PER_W, ROWS_PER_W)], idx_v)

    def chunk_idx(j, even):
      r = j >> 1
      if even:
        return idx_v.at[r, pl.ds(0, CA)]
      return idx_v.at[r, pl.ds(CA, CB)]

    def fire(j, b):
      even = (b % 2 == 0)
      n = CA if even else CB
      pltpu.async_copy(emb_hbm.at[chunk_idx(j, even)],
                       bufs[b].at[pl.ds(0, n)], sems[b])

    def wait(j, b):
      even = (b % 2 == 0)
      n = CA if even else CB
      pltpu.make_async_copy(emb_hbm.at[chunk_idx(j, even)],
                            bufs[b].at[pl.ds(0, n)], sems[b]).wait()

    def accum(buf, n, acc):
      def body(s, acc):
        a0, a1, a2, a3 = acc
        s4 = s * 4
        for u in range(4):
          r = s4 + u
          a0 = a0 + buf[r, pl.ds(0, 16)]
          a1 = a1 + buf[r, pl.ds(16, 16)]
          a2 = a2 + buf[r, pl.ds(32, 16)]
          a3 = a3 + buf[r, pl.ds(48, 16)]
        return (a0, a1, a2, a3)
      return lax.fori_loop(0, n // 4, body, acc)

    for b in range(NBUF):
      fire(b, b)

    def process(j0, last):
      for pair in range(NBUF // 2):
        zero = jnp.zeros((16,), jnp.float32)
        acc = (zero, zero, zero, zero)
        for h in range(2):
          b = 2 * pair + h
          j = j0 + b
          wait(j, b)
          acc = accum(bufs[b], CA if h == 0 else CB, acc)
          if not last:
            fire(j + NBUF, b)
        r = (j0 >> 1) + pair
        for k in range(4):
          out_v[r, pl.ds(k * 16, 16)] = acc[k]

    @pl.loop(0, CHUNKS_PER_W - NBUF, step=NBUF)
    def _(j0):
      process(j0, last=False)

    process(CHUNKS_PER_W - NBUF, last=True)

    pltpu.sync_copy(out_v, out_hbm.at[pl.ds(wid * ROWS_PER_W, ROWS_PER_W)])

  return pool(x, emb)


TBLK = 32768
NBLK = -(-1000000 // TBLK)
VPAD = NBLK * TBLK
HSHIFT = (TBLK // 2).bit_length() - 1


def _detile(embT):
  def body(in_ref, o_ref):
    arr = in_ref[...]
    a = arr[:, : TBLK // 2].T
    b = arr[:, TBLK // 2 :].T
    o_ref[:, :D] = a
    o_ref[:, D:] = b

  return pl.pallas_call(
      body,
      grid=(NBLK,),
      in_specs=[pl.BlockSpec((D, TBLK), lambda b: (0, b))],
      out_specs=pl.BlockSpec((TBLK // 2, 2 * D), lambda b: (b, 0)),
      out_shape=jax.ShapeDtypeStruct((VPAD // 2, 2 * D), jnp.float32),
  )(embT)


def _mlp(x, sums, emb0, W1, b1, W2, b2):
  BLK = 512

  def body(x_ref, s_ref, e0_ref, w1_ref, b1_ref, w2_ref, b2_ref, o_ref):
    xm = (x_ref[...] != 0).astype(jnp.float32)
    ln = jnp.sum(xm, axis=1, keepdims=True)
    avg = (s_ref[...] - (S - ln) * e0_ref[...]) / ln
    hdn = lax.dot_general(avg, w1_ref[...], (((1,), (1,)), ((), ())),
                          preferred_element_type=jnp.float32) + b1_ref[...]
    hdn = jnp.maximum(hdn, 0.0)
    logits = lax.dot_general(hdn, w2_ref[...], (((1,), (1,)), ((), ())),
                             preferred_element_type=jnp.float32) + b2_ref[...]
    m = jnp.max(logits, axis=1, keepdims=True)
    lse = m + jnp.log(jnp.sum(jnp.exp(logits - m), axis=1, keepdims=True))
    o_ref[...] = logits - lse

  return pl.pallas_call(
      body,
      grid=(B // BLK,),
      in_specs=[
          pl.BlockSpec((BLK, S), lambda i: (i, 0)),
          pl.BlockSpec((BLK, D), lambda i: (i, 0)),
          pl.BlockSpec((1, D), lambda i: (0, 0)),
          pl.BlockSpec((H, D), lambda i: (0, 0)),
          pl.BlockSpec((1, H), lambda i: (0, 0)),
          pl.BlockSpec((2, H), lambda i: (0, 0)),
          pl.BlockSpec((1, 2), lambda i: (0, 0)),
      ],
      out_specs=pl.BlockSpec((BLK, 2), lambda i: (i, 0)),
      out_shape=jax.ShapeDtypeStruct((B, 2), jnp.float32),
  )(x, sums, emb0, W1, b1, W2, b2)


def kernel(x, emb, W1, b1, W2, b2):
  x = x.astype(jnp.int32)
  emb_lin = _detile(emb.T).reshape(VPAD, D)
  e = x & (TBLK - 1)
  xr = (x - e) + ((e << 1) & (TBLK - 1)) + (e >> HSHIFT)
  sums = _sc_pool(xr, emb_lin)
  emb0 = lax.slice(emb, (0, 0), (1, D))
  return _mlp(x, sums, emb0, W1, b1.reshape(1, H), W2, b2.reshape(1, 2))
